# Initial kernel scaffold; baseline (speedup 1.0000x reference)
import jax
import jax.numpy as jnp
from jax.experimental import pallas as pl


def kernel(inputs, token_table, position_embedding):
    raise NotImplementedError("write your pallas kernel here")


# speedup vs baseline: 3.7494x; 3.7494x over previous
import functools

import jax
import jax.numpy as jnp
from jax import lax
from jax.experimental import pallas as pl
from jax.experimental.pallas import tpu as pltpu
from jax.experimental.pallas import tpu_sc as plsc

VOCAB = 100000
EMB = 128
B = 1024
S = 512
LANES = 16
NC = 2
NS = 16
NW = NC * NS
S_PER_W = S // NW
CHUNK = 128
NCHUNK = B // CHUNK


def _emb_body(idxT_hbm, table_hbm, pos_hbm, out_hbm,
              idx_v, rows_v, oidx_v, pos_v, gsem, ssem):
    wid = lax.axis_index("s") * NC + lax.axis_index("c")
    s0 = wid * S_PER_W

    def s_body(si, carry):
        s = s0 + si
        pltpu.sync_copy(pos_hbm.at[s], pos_v)
        pvecs = [pos_v[pl.ds(j * LANES, LANES)] for j in range(EMB // LANES)]

        def c_body(c, carry2):
            pltpu.sync_copy(idxT_hbm.at[s, pl.ds(c * CHUNK, CHUNK)], idx_v)
            pltpu.async_copy(table_hbm.at[idx_v], rows_v, gsem).wait()
            for k in range(CHUNK // LANES):
                lane = lax.iota(jnp.int32, LANES)
                oidx_v[pl.ds(k * LANES, LANES)] = (
                    lane * S + (c * CHUNK + k * LANES) * S + s)

            def t_body(t, carry3):
                for j in range(EMB // LANES):
                    sl = pl.ds(j * LANES, LANES)
                    rows_v[t, sl] = rows_v[t, sl] + pvecs[j]
                return carry3

            lax.fori_loop(0, CHUNK, t_body, 0)
            pltpu.async_copy(out_hbm.at[oidx_v], rows_v, ssem).wait()
            return carry2

        lax.fori_loop(0, NCHUNK, c_body, 0)
        return carry

    lax.fori_loop(0, S_PER_W, s_body, 0)


_emb = functools.partial(
    pl.kernel,
    out_type=jax.ShapeDtypeStruct((B * S, EMB), jnp.float32),
    mesh=plsc.VectorSubcoreMesh(core_axis_name="c", subcore_axis_name="s"),
    scratch_types=[
        pltpu.VMEM((CHUNK,), jnp.int32),
        pltpu.VMEM((CHUNK, EMB), jnp.float32),
        pltpu.VMEM((CHUNK,), jnp.int32),
        pltpu.VMEM((EMB,), jnp.float32),
        pltpu.SemaphoreType.DMA,
        pltpu.SemaphoreType.DMA,
    ],
)(_emb_body)


def kernel(inputs, token_table, position_embedding):
    idxT = jnp.transpose(inputs.astype(jnp.int32))
    out = _emb(idxT, token_table, position_embedding[:S])
    return out.reshape(B, S, EMB)


# speedup vs baseline: 9.0672x; 2.4183x over previous
import functools

import jax
import jax.numpy as jnp
from jax import lax
from jax.experimental import pallas as pl
from jax.experimental.pallas import tpu as pltpu
from jax.experimental.pallas import tpu_sc as plsc

VOCAB = 100000
EMB = 128
B = 1024
S = 512
LANES = 16
NC = 2
NS = 16
NW = NC * NS
S_PER_W = S // NW
CHUNK = 128
NCHUNK = B // CHUNK
PAIR = 2
NROUND = S_PER_W * NCHUNK // PAIR


def _emb_body(idxT_hbm, table_hbm, pos_hbm, out_hbm,
              idx_all, pos_all, rows_v, oidx_v, gsem, ssem):
    wid = lax.axis_index("s") * NC + lax.axis_index("c")
    s0 = wid * S_PER_W
    pltpu.sync_copy(idxT_hbm.at[pl.ds(s0, S_PER_W)], idx_all)
    pltpu.sync_copy(pos_hbm.at[pl.ds(s0, S_PER_W)], pos_all)

    def gather_of(r, half, b):
        t = PAIR * r + b
        si = t >> 3
        c = t & (NCHUNK - 1)
        slot = half * PAIR + b
        return pltpu.make_async_copy(
            table_hbm.at[idx_all.at[si, pl.ds(c * CHUNK, CHUNK)]],
            rows_v.at[slot], gsem.at[slot])

    def scatter_of(half, b):
        slot = half * PAIR + b
        return pltpu.make_async_copy(
            rows_v.at[slot], out_hbm.at[oidx_v.at[slot]], ssem.at[slot])

    def launch_half(r, half):
        for b in range(PAIR):
            gather_of(r, half, b).start()

    def wait_scatters(half):
        for b in range(PAIR):
            scatter_of(half, b).wait()

    def process_half(r, half):
        for b in range(PAIR):
            slot = half * PAIR + b
            t = PAIR * r + b
            si = t >> 3
            c = t & (NCHUNK - 1)
            s = s0 + si
            lane = lax.iota(jnp.int32, LANES)
            for k in range(CHUNK // LANES):
                oidx_v[slot, pl.ds(k * LANES, LANES)] = (
                    lane * S + (c * CHUNK + k * LANES) * S + s)
            pvecs = [pos_all[si, pl.ds(j * LANES, LANES)]
                     for j in range(EMB // LANES)]
            gather_of(r, half, b).wait()

            def t_body(tt, carry3):
                for j in range(EMB // LANES):
                    sl = pl.ds(j * LANES, LANES)
                    rows_v[slot, tt, sl] = rows_v[slot, tt, sl] + pvecs[j]
                return carry3

            lax.fori_loop(0, CHUNK, t_body, 0)
            scatter_of(half, b).start()

    def body(gg, carry):
        r0 = 2 * gg
        r1 = 2 * gg + 1

        @pl.when(gg >= 1)
        def _():
            wait_scatters(0)
            launch_half(r0, 0)
            process_half(r0 - 1, 1)
            wait_scatters(1)

        @pl.when(gg == 0)
        def _():
            launch_half(r0, 0)

        launch_half(r1, 1)
        process_half(r0, 0)
        return carry

    lax.fori_loop(0, NROUND // 2, body, 0)
    process_half(NROUND - 1, 1)
    wait_scatters(0)
    wait_scatters(1)


_emb = functools.partial(
    pl.kernel,
    out_type=jax.ShapeDtypeStruct((B * S, EMB), jnp.float32),
    mesh=plsc.VectorSubcoreMesh(core_axis_name="c", subcore_axis_name="s"),
    scratch_types=[
        pltpu.VMEM((S_PER_W, B), jnp.int32),
        pltpu.VMEM((S_PER_W, EMB), jnp.float32),
        pltpu.VMEM((2 * PAIR, CHUNK, EMB), jnp.float32),
        pltpu.VMEM((2 * PAIR, CHUNK), jnp.int32),
        pltpu.SemaphoreType.DMA((2 * PAIR,)),
        pltpu.SemaphoreType.DMA((2 * PAIR,)),
    ],
)(_emb_body)


def kernel(inputs, token_table, position_embedding):
    idxT = jnp.transpose(inputs.astype(jnp.int32))
    out = _emb(idxT, token_table, position_embedding[:S])
    return out.reshape(B, S, EMB)


# speedup vs baseline: 9.0806x; 1.0015x over previous
import functools

import jax
import jax.numpy as jnp
from jax import lax
from jax.experimental import pallas as pl
from jax.experimental.pallas import tpu as pltpu
from jax.experimental.pallas import tpu_sc as plsc

VOCAB = 100000
EMB = 128
B = 1024
S = 512
LANES = 16
NC = 2
NS = 16
NW = NC * NS
S_PER_W = S // NW
CHUNK = 128
NCHUNK = B // CHUNK
PAIR = 2
NROUND = S_PER_W * NCHUNK // PAIR


def _emb_body(idxT_hbm, table_hbm, pos_hbm, out_hbm,
              idx_all, pos_all, rows_v, oidx_v, gsem, ssem):
    wid = lax.axis_index("s") * NC + lax.axis_index("c")
    s0 = wid * S_PER_W
    pltpu.sync_copy(idxT_hbm.at[pl.ds(s0, S_PER_W)], idx_all)
    pltpu.sync_copy(pos_hbm.at[pl.ds(s0, S_PER_W)], pos_all)

    def gather_of(r, half, b):
        t = PAIR * r + b
        si = t >> 3
        c = t & (NCHUNK - 1)
        slot = half * PAIR + b
        return pltpu.make_async_copy(
            table_hbm.at[idx_all.at[si, pl.ds(c * CHUNK, CHUNK)]],
            rows_v.at[slot], gsem.at[slot])

    def scatter_of(half, b):
        slot = half * PAIR + b
        return pltpu.make_async_copy(
            rows_v.at[slot], out_hbm.at[oidx_v.at[slot]], ssem.at[slot])

    def launch_half(r, half):
        for b in range(PAIR):
            gather_of(r, half, b).start()

    def wait_scatters(half):
        for b in range(PAIR):
            scatter_of(half, b).wait()

    def process_half(r, half):
        for b in range(PAIR):
            slot = half * PAIR + b
            t = PAIR * r + b
            si = t >> 3
            c = t & (NCHUNK - 1)
            s = s0 + si
            lane = lax.iota(jnp.int32, LANES)
            for k in range(CHUNK // LANES):
                oidx_v[slot, pl.ds(k * LANES, LANES)] = (
                    lane * S + (c * CHUNK + k * LANES) * S + s)
            pvecs = [pos_all[si, pl.ds(j * LANES, LANES)]
                     for j in range(EMB // LANES)]
            gather_of(r, half, b).wait()

            @plsc.parallel_loop(0, CHUNK, unroll=4)
            def _(tt):
                for j in range(EMB // LANES):
                    plsc.addupdate(
                        rows_v.at[slot, tt, pl.ds(j * LANES, LANES)], pvecs[j])

            scatter_of(half, b).start()

    def body(gg, carry):
        r0 = 2 * gg
        r1 = 2 * gg + 1

        @pl.when(gg >= 1)
        def _():
            wait_scatters(0)
            launch_half(r0, 0)
            process_half(r0 - 1, 1)
            wait_scatters(1)

        @pl.when(gg == 0)
        def _():
            launch_half(r0, 0)

        launch_half(r1, 1)
        process_half(r0, 0)
        return carry

    lax.fori_loop(0, NROUND // 2, body, 0)
    process_half(NROUND - 1, 1)
    wait_scatters(0)
    wait_scatters(1)


_emb = functools.partial(
    pl.kernel,
    out_type=jax.ShapeDtypeStruct((B * S, EMB), jnp.float32),
    mesh=plsc.VectorSubcoreMesh(core_axis_name="c", subcore_axis_name="s"),
    scratch_types=[
        pltpu.VMEM((S_PER_W, B), jnp.int32),
        pltpu.VMEM((S_PER_W, EMB), jnp.float32),
        pltpu.VMEM((2 * PAIR, CHUNK, EMB), jnp.float32),
        pltpu.VMEM((2 * PAIR, CHUNK), jnp.int32),
        pltpu.SemaphoreType.DMA((2 * PAIR,)),
        pltpu.SemaphoreType.DMA((2 * PAIR,)),
    ],
)(_emb_body)


def kernel(inputs, token_table, position_embedding):
    idxT = jnp.transpose(inputs.astype(jnp.int32))
    out = _emb(idxT, token_table, position_embedding[:S])
    return out.reshape(B, S, EMB)
